# Initial kernel scaffold; baseline (speedup 1.0000x reference)
import functools
import numpy as np
import jax
import jax.numpy as jnp
from jax import lax
from jax.experimental import pallas as pl
from jax.experimental.pallas import tpu as pltpu
from jax.experimental.pallas import tpu_sc as plsc

NUM_CLASSES = 100
EMB_DIM = 768
OUT_DIM = 3072
BATCH = 8192

NC, NS = 2, 16
NW = NC * NS
ROWS_PW = BATCH // NW
CHUNK = 128
NCHUNK = ROWS_PW // CHUNK


def _build_upsample_matrix() -> np.ndarray:
    U = np.zeros((32, 16), np.float32)
    for i in range(32):
        c = (i + 0.5) / 2.0 - 0.5
        f = int(np.floor(c))
        t = c - f
        for (j, w) in ((f, 1.0 - t), (f + 1, t)):
            U[i, min(max(j, 0), 15)] += w
    K = np.einsum("ih,jw->hwij", U, U).reshape(256, 1024)
    M = np.zeros((EMB_DIM, OUT_DIM), np.float32)
    for c in range(3):
        M[c * 256:(c + 1) * 256, c * 1024:(c + 1) * 1024] = K
    return M


_M_BF16 = jnp.asarray(_build_upsample_matrix(), dtype=jnp.bfloat16)

_sc_mesh = plsc.VectorSubcoreMesh(core_axis_name="c", subcore_axis_name="s")


@functools.partial(
    pl.kernel,
    mesh=_sc_mesh,
    out_type=(
        jax.ShapeDtypeStruct((BATCH, EMB_DIM), jnp.float32),
        jax.ShapeDtypeStruct((BATCH, NUM_CLASSES), jnp.float32),
    ),
    scratch_types=[
        pltpu.VMEM((NCHUNK, CHUNK), jnp.int32),
        pltpu.VMEM((CHUNK, EMB_DIM), jnp.float32),
        pltpu.VMEM((ROWS_PW, NUM_CLASSES), jnp.float32),
        pltpu.SemaphoreType.DMA,
        pltpu.SemaphoreType.DMA,
    ],
)
def _sc_gather(idx_hbm, data_hbm, label_hbm, outd_hbm, outl_hbm,
               idx_v, rows_v, lab_v, sem_d, sem_l):
    wid = lax.axis_index("s") * NC + lax.axis_index("c")
    base = wid * ROWS_PW
    for j in range(NCHUNK):
        pltpu.sync_copy(idx_hbm.at[pl.ds(base + j * CHUNK, CHUNK)], idx_v.at[j])
    for j in range(NCHUNK):
        pltpu.async_copy(data_hbm.at[idx_v.at[j]], rows_v, sem_d).wait()
        pltpu.sync_copy(rows_v, outd_hbm.at[pl.ds(base + j * CHUNK, CHUNK)])
    for j in range(NCHUNK):
        pltpu.async_copy(
            label_hbm.at[idx_v.at[j]],
            lab_v.at[pl.ds(j * CHUNK, CHUNK)], sem_l).wait()
    pltpu.sync_copy(lab_v, outl_hbm.at[pl.ds(base, ROWS_PW)])


def _tc_upsample_body(x_ref, m_ref, o_ref):
    x = x_ref[...].astype(jnp.bfloat16)
    o_ref[...] = jnp.dot(x, m_ref[...], preferred_element_type=jnp.float32)


_TC_BLK = 512


def _tc_upsample(gathered):
    return pl.pallas_call(
        _tc_upsample_body,
        grid=(BATCH // _TC_BLK,),
        in_specs=[
            pl.BlockSpec((_TC_BLK, EMB_DIM), lambda i: (i, 0)),
            pl.BlockSpec((EMB_DIM, OUT_DIM), lambda i: (0, 0)),
        ],
        out_specs=pl.BlockSpec((_TC_BLK, OUT_DIM), lambda i: (i, 0)),
        out_shape=jax.ShapeDtypeStruct((BATCH, OUT_DIM), jnp.float32),
    )(gathered, _M_BF16)


@jax.jit
def kernel(indices, data_table, label_table):
    gathered, labels = _sc_gather(indices, data_table, label_table)
    imgs = _tc_upsample(gathered).reshape(BATCH, 3, 32, 32)
    return imgs, labels


# speedup vs baseline: 1.3629x; 1.3629x over previous
import functools
import numpy as np
import jax
import jax.numpy as jnp
from jax import lax
from jax.experimental import pallas as pl
from jax.experimental.pallas import tpu as pltpu
from jax.experimental.pallas import tpu_sc as plsc

NUM_CLASSES = 100
EMB_DIM = 768
OUT_DIM = 3072
BATCH = 8192

NC, NS = 2, 16
NW = NC * NS
ROWS_PW = BATCH // NW
CHUNK = 64
NCHUNK = ROWS_PW // CHUNK


def _build_upsample_matrix() -> np.ndarray:
    U = np.zeros((32, 16), np.float32)
    for i in range(32):
        c = (i + 0.5) / 2.0 - 0.5
        f = int(np.floor(c))
        t = c - f
        for (j, w) in ((f, 1.0 - t), (f + 1, t)):
            U[i, min(max(j, 0), 15)] += w
    K = np.einsum("ih,jw->hwij", U, U).reshape(256, 1024)
    M = np.zeros((EMB_DIM, OUT_DIM), np.float32)
    for c in range(3):
        M[c * 256:(c + 1) * 256, c * 1024:(c + 1) * 1024] = K
    return M


_M_NP = _build_upsample_matrix().astype(jnp.bfloat16)

_sc_mesh = plsc.VectorSubcoreMesh(core_axis_name="c", subcore_axis_name="s")


LAB_PAD = 128


@functools.partial(
    pl.kernel,
    mesh=_sc_mesh,
    out_type=(
        jax.ShapeDtypeStruct((BATCH, EMB_DIM), jnp.float32),
        jax.ShapeDtypeStruct((BATCH, LAB_PAD), jnp.float32),
    ),
    scratch_types=[
        [pltpu.VMEM((CHUNK,), jnp.int32) for _ in range(ROWS_PW // CHUNK)],
        pltpu.VMEM((CHUNK, EMB_DIM), jnp.float32),
        pltpu.VMEM((ROWS_PW, LAB_PAD), jnp.float32),
        pltpu.SemaphoreType.DMA,
        pltpu.SemaphoreType.DMA,
    ],
)
def _sc_gather(idx_hbm, data_hbm, labp_hbm, outd_hbm, outl_hbm,
               idx_bufs, rows_v, lab_v, sem_d, sem_l):
    wid = lax.axis_index("s") * NC + lax.axis_index("c")
    base = wid * ROWS_PW
    for j in range(NCHUNK):
        pltpu.sync_copy(idx_hbm.at[pl.ds(base + j * CHUNK, CHUNK)], idx_bufs[j])
    for j in range(NCHUNK):
        pltpu.async_copy(data_hbm.at[idx_bufs[j]], rows_v, sem_d).wait()
        pltpu.sync_copy(rows_v, outd_hbm.at[pl.ds(base + j * CHUNK, CHUNK)])
    for j in range(NCHUNK):
        pltpu.async_copy(labp_hbm.at[idx_bufs[j]],
                         lab_v.at[pl.ds(j * CHUNK, CHUNK)], sem_l).wait()
    pltpu.sync_copy(lab_v, outl_hbm.at[pl.ds(base, ROWS_PW)])


def _tc_upsample_body(x_ref, m_ref, o_ref):
    x = x_ref[...].astype(jnp.bfloat16)
    o_ref[...] = jnp.dot(x, m_ref[...], preferred_element_type=jnp.float32)


_TC_BLK = 512


def _tc_upsample(gathered):
    return pl.pallas_call(
        _tc_upsample_body,
        grid=(BATCH // _TC_BLK,),
        in_specs=[
            pl.BlockSpec((_TC_BLK, EMB_DIM), lambda i: (i, 0)),
            pl.BlockSpec((EMB_DIM, OUT_DIM), lambda i: (0, 0)),
        ],
        out_specs=pl.BlockSpec((_TC_BLK, OUT_DIM), lambda i: (i, 0)),
        out_shape=jax.ShapeDtypeStruct((BATCH, OUT_DIM), jnp.float32),
    )(gathered, jnp.asarray(_M_NP))


@jax.jit
def kernel(indices, data_table, label_table):
    labp = jnp.pad(label_table, ((0, 0), (0, LAB_PAD - NUM_CLASSES)))
    gathered, labels_p = _sc_gather(indices, data_table, labp)
    imgs = _tc_upsample(gathered).reshape(BATCH, 3, 32, 32)
    return imgs, labels_p[:, :NUM_CLASSES]


# speedup vs baseline: 1.7936x; 1.3160x over previous
import functools
import numpy as np
import jax
import jax.numpy as jnp
from jax import lax
from jax.experimental import pallas as pl
from jax.experimental.pallas import tpu as pltpu
from jax.experimental.pallas import tpu_sc as plsc

NUM_CLASSES = 100
NUM_EMB = 50000
EMB_DIM = 768
OUT_DIM = 3072
BATCH = 8192

NC, NS = 2, 16
NW = NC * NS
ROWS_PW = BATCH // NW
CHUNK = 64
NCHUNK = ROWS_PW // CHUNK


def _build_upsample_matrix() -> np.ndarray:
    U = np.zeros((32, 16), np.float32)
    for i in range(32):
        c = (i + 0.5) / 2.0 - 0.5
        f = int(np.floor(c))
        t = c - f
        for (j, w) in ((f, 1.0 - t), (f + 1, t)):
            U[i, min(max(j, 0), 15)] += w
    K = np.einsum("ih,jw->hwij", U, U).reshape(256, 1024)
    M = np.zeros((EMB_DIM, OUT_DIM), np.float32)
    for c in range(3):
        M[c * 256:(c + 1) * 256, c * 1024:(c + 1) * 1024] = K
    return M


_M_NP = _build_upsample_matrix().astype(jnp.bfloat16)

_sc_mesh = plsc.VectorSubcoreMesh(core_axis_name="c", subcore_axis_name="s")


LAB_PAD = 128


@functools.partial(
    pl.kernel,
    mesh=_sc_mesh,
    out_type=jax.ShapeDtypeStruct((BATCH, EMB_DIM), jnp.float32),
    scratch_types=[
        [pltpu.VMEM((CHUNK,), jnp.int32) for _ in range(NCHUNK)],
        pltpu.VMEM((CHUNK, EMB_DIM), jnp.float32),
        pltpu.SemaphoreType.DMA,
    ],
)
def _sc_gather_data(idx_hbm, data_hbm, outd_hbm, idx_bufs, rows_v, sem_d):
    wid = lax.axis_index("s") * NC + lax.axis_index("c")
    base = wid * ROWS_PW
    for j in range(NCHUNK):
        pltpu.sync_copy(idx_hbm.at[pl.ds(base + j * CHUNK, CHUNK)], idx_bufs[j])
    for j in range(NCHUNK):
        pltpu.async_copy(data_hbm.at[idx_bufs[j]], rows_v, sem_d).wait()
        pltpu.sync_copy(rows_v, outd_hbm.at[pl.ds(base + j * CHUNK, CHUNK)])


@functools.partial(
    pl.kernel,
    mesh=_sc_mesh,
    out_type=jax.ShapeDtypeStruct((BATCH, LAB_PAD), jnp.float32),
    scratch_types=[
        pltpu.VMEM((ROWS_PW,), jnp.int32),
        pltpu.VMEM((ROWS_PW, LAB_PAD), jnp.float32),
        pltpu.SemaphoreType.DMA,
    ],
)
def _sc_gather_labels(idx_hbm, labp_hbm, outl_hbm, idx_v, lab_v, sem_l):
    wid = lax.axis_index("s") * NC + lax.axis_index("c")
    base = wid * ROWS_PW
    pltpu.sync_copy(idx_hbm.at[pl.ds(base, ROWS_PW)], idx_v)
    pltpu.async_copy(labp_hbm.at[idx_v], lab_v, sem_l).wait()
    pltpu.sync_copy(lab_v, outl_hbm.at[pl.ds(base, ROWS_PW)])


def _tc_slice_body(x_ref, o_ref):
    o_ref[...] = x_ref[:, pl.ds(0, NUM_CLASSES)]


def _tc_slice_labels(labp_rows):
    return pl.pallas_call(
        _tc_slice_body,
        grid=(4,),
        in_specs=[pl.BlockSpec((BATCH // 4, LAB_PAD), lambda i: (i, 0))],
        out_specs=pl.BlockSpec((BATCH // 4, NUM_CLASSES), lambda i: (i, 0)),
        out_shape=jax.ShapeDtypeStruct((BATCH, NUM_CLASSES), jnp.float32),
    )(labp_rows)


def _tc_pad_body(x_ref, o_ref):
    o_ref[:, pl.ds(0, NUM_CLASSES)] = x_ref[...]


_PAD_BLK = 2000


def _tc_pad_labels(label_table):
    return pl.pallas_call(
        _tc_pad_body,
        grid=(NUM_EMB // _PAD_BLK,),
        in_specs=[pl.BlockSpec((_PAD_BLK, NUM_CLASSES), lambda i: (i, 0))],
        out_specs=pl.BlockSpec((_PAD_BLK, LAB_PAD), lambda i: (i, 0)),
        out_shape=jax.ShapeDtypeStruct((NUM_EMB, LAB_PAD), jnp.float32),
    )(label_table)


def _tc_upsample_body(x_ref, m_ref, o_ref):
    x = x_ref[...].astype(jnp.bfloat16)
    o_ref[...] = jnp.dot(x, m_ref[...], preferred_element_type=jnp.float32)


_TC_BLK = 512


def _tc_upsample(gathered):
    return pl.pallas_call(
        _tc_upsample_body,
        grid=(BATCH // _TC_BLK,),
        in_specs=[
            pl.BlockSpec((_TC_BLK, EMB_DIM), lambda i: (i, 0)),
            pl.BlockSpec((EMB_DIM, OUT_DIM), lambda i: (0, 0)),
        ],
        out_specs=pl.BlockSpec((_TC_BLK, OUT_DIM), lambda i: (i, 0)),
        out_shape=jax.ShapeDtypeStruct((BATCH, OUT_DIM), jnp.float32),
    )(gathered, jnp.asarray(_M_NP))


@jax.jit
def kernel(indices, data_table, label_table):
    labp = _tc_pad_labels(label_table)
    gathered = _sc_gather_data(indices, data_table)
    labels = _tc_slice_labels(_sc_gather_labels(indices, labp))
    imgs = _tc_upsample(gathered).reshape(BATCH, 3, 32, 32)
    return imgs, labels


# speedup vs baseline: 3.1642x; 1.7642x over previous
import functools
import numpy as np
import jax
import jax.numpy as jnp
from jax import lax
from jax.experimental import pallas as pl
from jax.experimental.pallas import tpu as pltpu
from jax.experimental.pallas import tpu_sc as plsc

NUM_CLASSES = 100
NUM_EMB = 50000
EMB_DIM = 768
OUT_DIM = 3072
BATCH = 8192

NC, NS = 2, 16
NW = NC * NS
ROWS_PW = BATCH // NW
CHUNK = 64
NCHUNK = ROWS_PW // CHUNK


def _build_upsample_matrix() -> np.ndarray:
    U = np.zeros((32, 16), np.float32)
    for i in range(32):
        c = (i + 0.5) / 2.0 - 0.5
        f = int(np.floor(c))
        t = c - f
        for (j, w) in ((f, 1.0 - t), (f + 1, t)):
            U[i, min(max(j, 0), 15)] += w
    K = np.einsum("ih,jw->hwij", U, U).reshape(256, 1024)
    M = np.zeros((EMB_DIM, OUT_DIM), np.float32)
    for c in range(3):
        M[c * 256:(c + 1) * 256, c * 1024:(c + 1) * 1024] = K
    return M


_MT_NP = np.ascontiguousarray(_build_upsample_matrix().T).astype(jnp.bfloat16)

_sc_mesh = plsc.VectorSubcoreMesh(core_axis_name="c", subcore_axis_name="s")


LAB_PAD = 128


@functools.partial(
    pl.kernel,
    mesh=_sc_mesh,
    out_type=jax.ShapeDtypeStruct((BATCH, EMB_DIM), jnp.float32),
    scratch_types=[
        [pltpu.VMEM((CHUNK,), jnp.int32) for _ in range(NCHUNK)],
        pltpu.VMEM((CHUNK, EMB_DIM), jnp.float32),
        pltpu.SemaphoreType.DMA,
    ],
)
def _sc_gather_data(idx_hbm, data_hbm, outd_hbm, idx_bufs, rows_v, sem_d):
    wid = lax.axis_index("s") * NC + lax.axis_index("c")
    base = wid * ROWS_PW
    for j in range(NCHUNK):
        pltpu.sync_copy(idx_hbm.at[pl.ds(base + j * CHUNK, CHUNK)], idx_bufs[j])
    for j in range(NCHUNK):
        pltpu.async_copy(data_hbm.at[idx_bufs[j]], rows_v, sem_d).wait()
        pltpu.sync_copy(rows_v, outd_hbm.at[pl.ds(base + j * CHUNK, CHUNK)])


@functools.partial(
    pl.kernel,
    mesh=_sc_mesh,
    out_type=jax.ShapeDtypeStruct((BATCH, LAB_PAD), jnp.float32),
    scratch_types=[
        pltpu.VMEM((ROWS_PW,), jnp.int32),
        pltpu.VMEM((ROWS_PW, LAB_PAD), jnp.float32),
        pltpu.SemaphoreType.DMA,
    ],
)
def _sc_gather_labels(idx_hbm, labp_hbm, outl_hbm, idx_v, lab_v, sem_l):
    wid = lax.axis_index("s") * NC + lax.axis_index("c")
    base = wid * ROWS_PW
    pltpu.sync_copy(idx_hbm.at[pl.ds(base, ROWS_PW)], idx_v)
    pltpu.async_copy(labp_hbm.at[idx_v], lab_v, sem_l).wait()
    pltpu.sync_copy(lab_v, outl_hbm.at[pl.ds(base, ROWS_PW)])


def _tc_slice_body(x_ref, o_ref):
    o_ref[...] = x_ref[:, pl.ds(0, NUM_CLASSES)]


def _tc_slice_labels(labp_rows):
    return pl.pallas_call(
        _tc_slice_body,
        grid=(4,),
        in_specs=[pl.BlockSpec((BATCH // 4, LAB_PAD), lambda i: (i, 0))],
        out_specs=pl.BlockSpec((BATCH // 4, NUM_CLASSES), lambda i: (i, 0)),
        out_shape=jax.ShapeDtypeStruct((BATCH, NUM_CLASSES), jnp.float32),
    )(labp_rows)


def _tc_padT_body(xt_ref, o_ref, scr_ref):
    scr_ref[pl.ds(0, NUM_CLASSES), :] = xt_ref[...]
    o_ref[...] = scr_ref[...].T


_PAD_BLK = 2048


def _tc_pad_labels(label_table_t):
    return pl.pallas_call(
        _tc_padT_body,
        grid=(pl.cdiv(NUM_EMB, _PAD_BLK),),
        in_specs=[pl.BlockSpec((NUM_CLASSES, _PAD_BLK), lambda i: (0, i))],
        out_specs=pl.BlockSpec((_PAD_BLK, LAB_PAD), lambda i: (i, 0)),
        out_shape=jax.ShapeDtypeStruct((NUM_EMB, LAB_PAD), jnp.float32),
        scratch_shapes=[pltpu.VMEM((LAB_PAD, _PAD_BLK), jnp.float32)],
    )(label_table_t)


def _tc_upsample_body(x_ref, mt_ref, o_ref):
    x = x_ref[...].astype(jnp.bfloat16)
    o_ref[...] = lax.dot_general(
        mt_ref[...], x, (((1,), (1,)), ((), ())),
        preferred_element_type=jnp.float32)


_TC_BLK = 512


def _tc_upsample(gathered):
    return pl.pallas_call(
        _tc_upsample_body,
        grid=(BATCH // _TC_BLK,),
        in_specs=[
            pl.BlockSpec((_TC_BLK, EMB_DIM), lambda i: (i, 0)),
            pl.BlockSpec((OUT_DIM, EMB_DIM), lambda i: (0, 0)),
        ],
        out_specs=pl.BlockSpec((OUT_DIM, _TC_BLK), lambda i: (0, i)),
        out_shape=jax.ShapeDtypeStruct((OUT_DIM, BATCH), jnp.float32),
    )(gathered, jnp.asarray(_MT_NP))


@jax.jit
def kernel(indices, data_table, label_table):
    labp = _tc_pad_labels(label_table.T)
    gathered = _sc_gather_data(indices, data_table)
    labels = _tc_slice_labels(_sc_gather_labels(indices, labp))
    imgs_t = _tc_upsample(gathered)
    imgs = imgs_t.reshape(3, 32, 32, BATCH).transpose(3, 0, 1, 2)
    return imgs, labels
